# Initial kernel scaffold; baseline (speedup 1.0000x reference)
import jax
import jax.numpy as jnp
from jax.experimental import pallas as pl


def kernel(node_type, node_value, edge_index, batch, edge_id, c1a_W1, c1a_b1, c1a_W2, c1a_b2, c1b_W1, c1b_b1, c1b_W2, c1b_b2, p1_W, p1_b, c2a_W1, c2a_b1, c2a_W2, c2a_b2, c2b_W1, c2b_b1, c2b_W2, c2b_b2, p2_W, p2_b, out_W, out_b):
    raise NotImplementedError("write your pallas kernel here")


# speedup vs baseline: 1.0014x; 1.0014x over previous
import functools

import jax
import jax.numpy as jnp
from jax.experimental import pallas as pl

N = 100000
E = 3200000
G = 1024
IC = 16


def _dense_block(x_ref, w_ref, b_ref, o_ref, *, pre_silu, post_silu):
    x = x_ref[...]
    if pre_silu:
        x = x * jax.nn.sigmoid(x)
    y = jnp.dot(x, w_ref[...], preferred_element_type=jnp.float32) + b_ref[...]
    if post_silu:
        y = y * jax.nn.sigmoid(y)
    o_ref[...] = y


def _dense(x, W, b, pre_silu=False, post_silu=False, block=4000):
    n, f = x.shape
    k = W.shape[1]
    b2d = b.reshape(1, k)
    grid = n // block
    assert n % block == 0
    return pl.pallas_call(
        functools.partial(_dense_block, pre_silu=pre_silu, post_silu=post_silu),
        grid=(grid,),
        in_specs=[
            pl.BlockSpec((block, f), lambda i: (i, 0)),
            pl.BlockSpec((f, k), lambda i: (0, 0)),
            pl.BlockSpec((1, k), lambda i: (0, 0)),
        ],
        out_specs=pl.BlockSpec((block, k), lambda i: (i, 0)),
        out_shape=jax.ShapeDtypeStruct((n, k), jnp.float32),
    )(x, W, b2d)


def _node_tables(h, W1, b1, pre_silu):
    f = h.shape[1]
    Wt, Wb = W1[:f], W1[f:]
    A = _dense(h, Wt - Wb, b1, pre_silu=pre_silu)
    B = _dense(h, Wb, jnp.zeros_like(b1), pre_silu=pre_silu)
    return A, B


def _edge_conv(A, B, src, dst, W2, b2):
    z = jnp.take(A, dst, axis=0) + jnp.take(B, src, axis=0)
    m = _dense(z, W2, b2, pre_silu=True)
    out = jax.ops.segment_max(m, dst, num_segments=N)
    return jnp.where(jnp.isneginf(out), jnp.zeros_like(out), out)


def _att_agg(x, batch, gW, gb):
    gate = _dense(x, gW, gb)
    gmax = jax.ops.segment_max(gate, batch, num_segments=G)
    gmax = jnp.where(jnp.isneginf(gmax), jnp.zeros_like(gmax), gmax)
    e = jnp.exp(gate - jnp.take(gmax, batch, axis=0))
    s = jax.ops.segment_sum(e, batch, num_segments=G)
    attn = e / (jnp.take(s, batch, axis=0) + 1e-16)
    return jax.ops.segment_sum(attn * x, batch, num_segments=G)


def kernel(node_type, node_value, edge_index, batch, edge_id,
           c1a_W1, c1a_b1, c1a_W2, c1a_b2, c1b_W1, c1b_b1, c1b_W2, c1b_b2,
           p1_W, p1_b,
           c2a_W1, c2a_b1, c2a_W2, c2a_b2, c2b_W1, c2b_b1, c2b_W2, c2b_b2,
           p2_W, p2_b, out_W, out_b):
    src = edge_index[0]
    dst = edge_index[1]
    x = jnp.stack([node_type, node_value], axis=1)

    A, B = _node_tables(x, c1a_W1, c1a_b1, pre_silu=False)
    h = _edge_conv(A, B, src, dst, c1a_W2, c1a_b2)
    A, B = _node_tables(h, c1b_W1, c1b_b1, pre_silu=True)
    h = _edge_conv(A, B, src, dst, c1b_W2, c1b_b2)

    pooled = _att_agg(h, batch, p1_W, p1_b)
    x2 = jnp.concatenate([x, jnp.take(pooled, batch, axis=0)], axis=1)

    A, B = _node_tables(x2, c2a_W1, c2a_b1, pre_silu=False)
    h2 = _edge_conv(A, B, src, dst, c2a_W2, c2a_b2)
    A, B = _node_tables(h2, c2b_W1, c2b_b1, pre_silu=True)
    h2 = _edge_conv(A, B, src, dst, c2b_W2, c2b_b2)

    out = _att_agg(h2, batch, p2_W, p2_b)
    return _dense(out, out_W, out_b, block=G)
